# Initial kernel scaffold; baseline (speedup 1.0000x reference)
import jax
import jax.numpy as jnp
from jax.experimental import pallas as pl


def kernel(labels, mean, log_var):
    raise NotImplementedError("write your pallas kernel here")


# speedup vs baseline: 102.1828x; 102.1828x over previous
import functools

import jax
import jax.numpy as jnp
from jax import lax
from jax.experimental import pallas as pl
from jax.experimental.pallas import tpu as pltpu
from jax.experimental.pallas import tpu_sc as plsc

_NUM_CORES = 2
_NUM_SUBCORES = 16
_NUM_WORKERS = _NUM_CORES * _NUM_SUBCORES
_LANES = 16


@functools.lru_cache(maxsize=None)
def _build(B, D, C):
    total = B * D
    assert total % _NUM_WORKERS == 0
    chunk = total // _NUM_WORKERS
    assert chunk % _LANES == 0 and chunk % 8 == 0
    n_iters = chunk // _LANES
    assert chunk % D == 0

    mesh = plsc.VectorSubcoreMesh(core_axis_name="c", subcore_axis_name="s")

    @functools.partial(
        pl.kernel,
        out_type=(
            jax.ShapeDtypeStruct((total,), jnp.float32),
            jax.ShapeDtypeStruct((total,), jnp.float32),
        ),
        mesh=mesh,
        compiler_params=pltpu.CompilerParams(needs_layout_passes=False),
        scratch_types=[
            pltpu.VMEM((chunk,), jnp.int32),
            pltpu.VMEM((D * C,), jnp.float32),
            pltpu.VMEM((D * C,), jnp.float32),
            pltpu.VMEM((chunk,), jnp.float32),
            pltpu.VMEM((chunk,), jnp.float32),
        ],
    )
    def gather_kernel(labels_hbm, mean_hbm, lv_hbm, outm_hbm, outlv_hbm,
                      labels_v, mean_v, lv_v, outm_v, outlv_v):
        wid = lax.axis_index("c") * _NUM_SUBCORES + lax.axis_index("s")
        base = wid * chunk
        pltpu.sync_copy(labels_hbm.at[pl.ds(base, chunk)], labels_v)
        pltpu.sync_copy(mean_hbm, mean_v)
        pltpu.sync_copy(lv_hbm, lv_v)

        lane = lax.iota(jnp.int32, _LANES)

        def body(i, carry):
            off = i * _LANES
            lbl = labels_v[pl.ds(off, _LANES)]
            idx = lax.rem(lane + off, D) * C + lbl
            outm_v[pl.ds(off, _LANES)] = plsc.load_gather(mean_v, [idx])
            outlv_v[pl.ds(off, _LANES)] = plsc.load_gather(lv_v, [idx])
            return carry

        lax.fori_loop(0, n_iters, body, 0)

        pltpu.sync_copy(outm_v, outm_hbm.at[pl.ds(base, chunk)])
        pltpu.sync_copy(outlv_v, outlv_hbm.at[pl.ds(base, chunk)])

    return gather_kernel


def kernel(labels, mean, log_var):
    B, D = labels.shape
    C = mean.shape[1]
    labels_flat = labels.astype(jnp.int32).reshape(-1)
    gather_kernel = _build(B, D, C)
    outm, outlv = gather_kernel(labels_flat, mean.reshape(-1), log_var.reshape(-1))
    return outm.reshape(B, D), outlv.reshape(B, D)


# speedup vs baseline: 146.9391x; 1.4380x over previous
import functools

import jax
import jax.numpy as jnp
from jax import lax
from jax.experimental import pallas as pl
from jax.experimental.pallas import tpu as pltpu
from jax.experimental.pallas import tpu_sc as plsc

_NUM_CORES = 2
_NUM_SUBCORES = 16
_NUM_WORKERS = _NUM_CORES * _NUM_SUBCORES
_LANES = 16
_CHUNK_ROWS = 128


@functools.lru_cache(maxsize=None)
def _build(B, D, C):
    assert B % _NUM_WORKERS == 0
    rows = B // _NUM_WORKERS
    ch = min(_CHUNK_ROWS, rows)
    assert rows % ch == 0
    n_chunks = rows // ch
    assert _LANES <= D <= 2 * _LANES

    mesh = plsc.VectorSubcoreMesh(core_axis_name="c", subcore_axis_name="s")

    @functools.partial(
        pl.kernel,
        out_type=(
            jax.ShapeDtypeStruct((B, D), jnp.float32),
            jax.ShapeDtypeStruct((B, D), jnp.float32),
        ),
        mesh=mesh,
        compiler_params=pltpu.CompilerParams(needs_layout_passes=False),
        scratch_types=[
            pltpu.VMEM((ch, D), jnp.int32),
            pltpu.VMEM((D, C), jnp.float32),
            pltpu.VMEM((D, C), jnp.float32),
            pltpu.VMEM((ch, D), jnp.float32),
            pltpu.VMEM((ch, D), jnp.float32),
        ],
    )
    def gather_kernel(labels_hbm, mean_hbm, lv_hbm, outm_hbm, outlv_hbm,
                      labels_v, mean_v, lv_v, outm_v, outlv_v):
        wid = lax.axis_index("c") * _NUM_SUBCORES + lax.axis_index("s")
        row0 = wid * rows
        pltpu.sync_copy(mean_hbm, mean_v)
        pltpu.sync_copy(lv_hbm, lv_v)
        d_lo = lax.iota(jnp.int32, _LANES)
        d_hi = d_lo + (D - _LANES)

        def chunk_body(ci, carry):
            base = row0 + ci * ch
            pltpu.sync_copy(labels_hbm.at[pl.ds(base, ch)], labels_v)

            def row_body(r, c2):
                lbl_lo = labels_v[r, pl.ds(0, _LANES)]
                lbl_hi = labels_v[r, pl.ds(D - _LANES, _LANES)]
                outm_v[r, pl.ds(0, _LANES)] = plsc.load_gather(
                    mean_v, [d_lo, lbl_lo])
                outm_v[r, pl.ds(D - _LANES, _LANES)] = plsc.load_gather(
                    mean_v, [d_hi, lbl_hi])
                outlv_v[r, pl.ds(0, _LANES)] = plsc.load_gather(
                    lv_v, [d_lo, lbl_lo])
                outlv_v[r, pl.ds(D - _LANES, _LANES)] = plsc.load_gather(
                    lv_v, [d_hi, lbl_hi])
                return c2

            lax.fori_loop(0, ch, row_body, 0)
            pltpu.sync_copy(outm_v, outm_hbm.at[pl.ds(base, ch)])
            pltpu.sync_copy(outlv_v, outlv_hbm.at[pl.ds(base, ch)])
            return carry

        lax.fori_loop(0, n_chunks, chunk_body, 0)

    return gather_kernel


def kernel(labels, mean, log_var):
    B, D = labels.shape
    C = mean.shape[1]
    gather_kernel = _build(B, D, C)
    return gather_kernel(labels.astype(jnp.int32), mean, log_var)
